# baseline (speedup 1.0000x reference)
import functools

import jax
import jax.numpy as jnp
from jax import lax
from jax.experimental import pallas as pl
from jax.experimental.pallas import tpu as pltpu
from jax.experimental.pallas import tpu_sc as plsc

EMBED_DIM = 32
NUM_WORKERS = 32
CHUNK = 128
NSLOT = 2
LANES = 16
QPR = 128 // EMBED_DIM
RPC = CHUNK * EMBED_DIM // 128


@functools.lru_cache(maxsize=None)
def _make_gather(b_total):
    assert b_total % (NUM_WORKERS * CHUNK) == 0
    nchunk_total = b_total // CHUNK
    nchunk = nchunk_total // NUM_WORKERS
    nouter = nchunk // NSLOT
    assert nchunk == nouter * NSLOT

    mesh = plsc.VectorSubcoreMesh(core_axis_name="c", subcore_axis_name="s")

    @functools.partial(
        pl.kernel,
        mesh=mesh,
        out_type=jax.ShapeDtypeStruct(
            (b_total * EMBED_DIM // 128, 128), jnp.float32),
        scratch_types=[
            pltpu.VMEM((nchunk, CHUNK), jnp.int32),
            pltpu.VMEM((NSLOT, CHUNK), jnp.int32),
            pltpu.VMEM((NSLOT, CHUNK, 128), jnp.float32),
            pltpu.VMEM((NSLOT, RPC, 128), jnp.float32),
            pltpu.SemaphoreType.DMA,
            pltpu.SemaphoreType.DMA,
            pltpu.SemaphoreType.DMA,
            pltpu.SemaphoreType.DMA,
        ],
        compiler_params=pltpu.CompilerParams(
            use_tc_tiling_on_sc=False, needs_layout_passes=False),
    )
    def gather_kernel(table_hbm, idx_hbm, out_hbm, idx_v, idx4_v, rows_v,
                      pack_v, gsem0, gsem1, ssem0, ssem1):
        gsems = (gsem0, gsem1)
        ssems = (ssem0, ssem1)
        wid = lax.axis_index("s") * 2 + lax.axis_index("c")
        base = wid * nchunk
        pltpu.sync_copy(idx_hbm.at[pl.ds(base, nchunk)], idx_v)

        iota = lax.iota(jnp.int32, LANES)

        def fill_idx4(chunk, b):
            for blk in range(CHUNK // LANES):
                i0 = blk * LANES
                v = idx_v[chunk, pl.ds(i0, LANES)]
                idx4_v[b, pl.ds(i0, LANES)] = v >> 2

        def start_gather(b):
            pltpu.make_async_copy(
                table_hbm.at[idx4_v.at[b]], rows_v.at[b], gsems[b]).start()

        def wait_gather(b):
            pltpu.make_async_copy(
                table_hbm.at[idx4_v.at[b]], rows_v.at[b], gsems[b]).wait()

        def store_desc(chunk, b):
            return pltpu.make_async_copy(
                pack_v.at[b],
                out_hbm.at[pl.ds((base + chunk) * RPC, RPC)], ssems[b])

        def select(chunk, b):
            rows = rows_v.at[b]
            pack = pack_v.at[b]
            for blk in range(CHUNK // LANES):
                i0 = blk * LANES
                idxv = idx_v[chunk, pl.ds(i0, LANES)]
                q0 = (idxv & 3) << 5
                e0 = (iota + i0) << 5
                row_i = iota + i0
                for c in range(EMBED_DIM):
                    e = e0 + c
                    vals = plsc.load_gather(rows, [row_i, q0 + c])
                    plsc.store_scatter(pack, [e >> 7, e & 127], vals)

        for b in range(NSLOT):
            fill_idx4(b, b)
            start_gather(b)

        def body(outer, carry):
            for b in range(NSLOT):
                chunk = outer * NSLOT + b
                wait_gather(b)

                @pl.when(outer > 0)
                def _():
                    store_desc(chunk - NSLOT, b).wait()

                select(chunk, b)
                store_desc(chunk, b).start()

                @pl.when(outer < nouter - 1)
                def _():
                    fill_idx4(chunk + NSLOT, b)
                    start_gather(b)

            return carry

        lax.fori_loop(0, nouter, body, 0)

        for b in range(NSLOT):
            store_desc(nchunk - NSLOT + b, b).wait()

    return gather_kernel


def kernel(inputs, embeddings):
    idx = inputs.astype(jnp.int32).reshape(-1, CHUNK)
    table = embeddings.reshape(-1, 128)
    out = _make_gather(idx.size)(table, idx)
    return out.reshape(inputs.shape + (embeddings.shape[-1],))


# speedup vs baseline: 1.6036x; 1.6036x over previous
import functools

import jax
import jax.numpy as jnp
from jax import lax
from jax.experimental import pallas as pl
from jax.experimental.pallas import tpu as pltpu
from jax.experimental.pallas import tpu_sc as plsc

EMBED_DIM = 32
NUM_WORKERS = 32
CHUNK = 128
NSLOT = 2
LANES = 16
QPR = 128 // EMBED_DIM
RPC = CHUNK * EMBED_DIM // 128


@functools.lru_cache(maxsize=None)
def _make_gather(b_total):
    assert b_total % (NUM_WORKERS * CHUNK) == 0
    nchunk_total = b_total // CHUNK
    nchunk = nchunk_total // NUM_WORKERS
    nouter = nchunk // NSLOT
    assert nchunk == nouter * NSLOT

    mesh = plsc.VectorSubcoreMesh(core_axis_name="c", subcore_axis_name="s")

    @functools.partial(
        pl.kernel,
        mesh=mesh,
        out_type=jax.ShapeDtypeStruct(
            (b_total * EMBED_DIM // 128, 128), jnp.float32),
        scratch_types=[
            pltpu.VMEM((nchunk, CHUNK), jnp.int32),
            pltpu.VMEM((NSLOT, CHUNK), jnp.int32),
            pltpu.VMEM((NSLOT, CHUNK, 128), jnp.float32),
            pltpu.VMEM((NSLOT, RPC, 128), jnp.float32),
            pltpu.SemaphoreType.DMA,
            pltpu.SemaphoreType.DMA,
            pltpu.SemaphoreType.DMA,
            pltpu.SemaphoreType.DMA,
        ],
        compiler_params=pltpu.CompilerParams(
            use_tc_tiling_on_sc=False, needs_layout_passes=False),
    )
    def gather_kernel(table_hbm, idx_hbm, out_hbm, idx_v, idx4_v, rows_v,
                      pack_v, gsem0, gsem1, ssem0, ssem1):
        gsems = (gsem0, gsem1)
        ssems = (ssem0, ssem1)
        wid = lax.axis_index("s") * 2 + lax.axis_index("c")
        base = wid * nchunk
        pltpu.sync_copy(idx_hbm.at[pl.ds(base, nchunk)], idx_v)

        iota = lax.iota(jnp.int32, LANES)

        def fill_idx4(chunk, b):
            for blk in range(CHUNK // LANES):
                i0 = blk * LANES
                v = idx_v[chunk, pl.ds(i0, LANES)]
                idx4_v[b, pl.ds(i0, LANES)] = v >> 2

        def start_gather(b):
            pltpu.make_async_copy(
                table_hbm.at[idx4_v.at[b]], rows_v.at[b], gsems[b]).start()

        def wait_gather(b):
            pltpu.make_async_copy(
                table_hbm.at[idx4_v.at[b]], rows_v.at[b], gsems[b]).wait()

        def store_desc(chunk, b):
            return pltpu.make_async_copy(
                pack_v.at[b],
                out_hbm.at[pl.ds((base + chunk) * RPC, RPC)], ssems[b])

        def select(chunk, b):
            for blk in range(CHUNK // LANES):
                offv = (idx_v[chunk, pl.ds(blk * LANES, LANES)] & 3) << 5
                for t in range(LANES):
                    i = blk * LANES + t
                    off = offv[t]
                    r, c0 = i >> 2, (i & 3) << 5
                    for h in range(EMBED_DIM // LANES):
                        pack_v[b, r, pl.ds(c0 + h * LANES, LANES)] = (
                            rows_v[b, i, pl.ds(off + h * LANES, LANES)])

        for b in range(NSLOT):
            fill_idx4(b, b)
            start_gather(b)

        def body(outer, carry):
            for b in range(NSLOT):
                chunk = outer * NSLOT + b
                wait_gather(b)

                @pl.when(outer > 0)
                def _():
                    store_desc(chunk - NSLOT, b).wait()

                select(chunk, b)
                store_desc(chunk, b).start()

                @pl.when(outer < nouter - 1)
                def _():
                    fill_idx4(chunk + NSLOT, b)
                    start_gather(b)

            return carry

        lax.fori_loop(0, nouter, body, 0)

        for b in range(NSLOT):
            store_desc(nchunk - NSLOT + b, b).wait()

    return gather_kernel


def kernel(inputs, embeddings):
    idx = inputs.astype(jnp.int32).reshape(-1, CHUNK)
    table = embeddings.reshape(-1, 128)
    out = _make_gather(idx.size)(table, idx)
    return out.reshape(inputs.shape + (embeddings.shape[-1],))
